# baseline (speedup 1.0000x reference)
import functools

import jax
import jax.numpy as jnp
from jax import lax
from jax.experimental import pallas as pl
from jax.experimental.pallas import tpu as pltpu
from jax.experimental.pallas import tpu_sc as plsc

_Q = 0.7
_K = 0.8
_C = (1.0 - _K**_Q) / _Q

_B = 16
_HB = _B // 2
_NCLS = 21
_H = 256
_W = 256
_ROWS = 2048
_N = _B * _H * _W

_NC = 2
_NS = 16
_NW = _NC * _NS
_HSL = _H // _NW


def _sc_gather_body(table_ref, idx_ref, out_ref, idx_v, buf_v, sem, *, base):
    wid = lax.axis_index("s") * _NC + lax.axis_index("c")
    pltpu.sync_copy(idx_ref.at[pl.ds(base, _HB)], idx_v)
    h0 = wid * _HSL
    pltpu.async_copy(table_ref.at[idx_v, pl.ds(h0, _HSL)], buf_v, sem).wait()
    pltpu.sync_copy(buf_v, out_ref.at[:, pl.ds(h0, _HSL)])


def _sc_gather_half(table, idx, base):
    mesh = plsc.VectorSubcoreMesh(
        core_axis_name="c", subcore_axis_name="s",
        num_cores=_NC, num_subcores=_NS)
    return pl.kernel(
        functools.partial(_sc_gather_body, base=base),
        out_type=jax.ShapeDtypeStruct((_HB, _H, _W), jnp.float32),
        mesh=mesh,
        scratch_types=[
            pltpu.VMEM((_HB,), jnp.int32),
            pltpu.VMEM((_HB, _HSL, _W), jnp.float32),
            pltpu.SemaphoreType.DMA,
        ],
        name=f"sc_gather_b{base}",
    )(table, idx)


def _tc_loss_body(logits_ref, targets_ref, w_ref, acc_ref, out_ref, *, init):
    b = pl.program_id(0)
    r = pl.program_id(1)
    l = logits_ref[0]
    t = targets_ref[0]
    w = w_ref[0]
    m = jnp.max(l, axis=0)
    e = jnp.exp(l - m[None])
    s = jnp.sum(e, axis=0)
    cls = lax.broadcasted_iota(jnp.int32, l.shape, 0)
    lt = jnp.sum(jnp.where(cls == t[None], l, 0.0), axis=0)
    log_yg = (lt - m) - jnp.log(s)
    pow_q = jnp.exp(_Q * log_yg)
    term = (1.0 - pow_q) * (1.0 / _Q) - _C
    partial = jnp.sum(term * w) * (1.0 / _N)

    if init:
        @pl.when((b == 0) & (r == 0))
        def _init():
            out_ref[0, 0] = 0.0
    del acc_ref

    out_ref[0, 0] += partial


def _tc_loss_half(logits, targets, w_half, acc, base, init, block_r=128):
    nr = _H // block_r
    return pl.pallas_call(
        functools.partial(_tc_loss_body, init=init),
        grid=(_HB, nr),
        in_specs=[
            pl.BlockSpec((1, _NCLS, block_r, _W),
                         lambda b, r: (base + b, 0, r, 0)),
            pl.BlockSpec((1, block_r, _W), lambda b, r: (base + b, r, 0)),
            pl.BlockSpec((1, block_r, _W), lambda b, r: (b, r, 0)),
            pl.BlockSpec((1, 1), lambda b, r: (0, 0),
                         memory_space=pltpu.SMEM),
        ],
        out_specs=pl.BlockSpec((1, 1), lambda b, r: (0, 0),
                               memory_space=pltpu.SMEM),
        out_shape=jax.ShapeDtypeStruct((1, 1), jnp.float32),
        input_output_aliases={3: 0},
        name=f"tc_loss_b{base}",
    )(logits, targets, w_half, acc)


@jax.jit
def _loss(logits, weight, targets, indexes):
    table = weight.reshape(_ROWS, _H, _W)
    targets3 = targets.reshape(_B, _H, _W)
    w_a = _sc_gather_half(table, indexes, 0)
    w_b = _sc_gather_half(table, indexes, _HB)
    acc0 = jnp.zeros((1, 1), jnp.float32)
    out = _tc_loss_half(logits, targets3, w_a, acc0, 0, init=True)
    out = _tc_loss_half(logits, targets3, w_b, out, _HB, init=False)
    return out[0, 0]


def kernel(logits, weight, targets, indexes):
    return _loss(logits, weight, targets, indexes)


# speedup vs baseline: 1.1053x; 1.1053x over previous
import functools

import jax
import jax.numpy as jnp
from jax import lax
from jax.experimental import pallas as pl
from jax.experimental.pallas import tpu as pltpu
from jax.experimental.pallas import tpu_sc as plsc

_Q = 0.7
_K = 0.8
_C = (1.0 - _K**_Q) / _Q

_B = 16
_NCLS = 21
_H = 256
_W = 256
_ROWS = 2048
_N = _B * _H * _W

_NC = 2
_NS = 16
_NW = _NC * _NS
_CHUNK = 2048
_CPR = (_H * _W) // _CHUNK


_HSL = _H // _NW


def _sc_gather_body(table_ref, idx_ref, out_ref, idx_v, buf_v, sem):
    wid = lax.axis_index("s") * _NC + lax.axis_index("c")
    pltpu.sync_copy(idx_ref, idx_v)
    h0 = wid * _HSL
    pltpu.async_copy(table_ref.at[idx_v, pl.ds(h0, _HSL)], buf_v, sem).wait()
    pltpu.sync_copy(buf_v, out_ref.at[:, pl.ds(h0, _HSL)])


@jax.jit
def _sc_gather(table, idx):
    mesh = plsc.VectorSubcoreMesh(
        core_axis_name="c", subcore_axis_name="s",
        num_cores=_NC, num_subcores=_NS)
    return pl.kernel(
        _sc_gather_body,
        out_type=jax.ShapeDtypeStruct((_B, _H, _W), jnp.float32),
        mesh=mesh,
        scratch_types=[
            pltpu.VMEM((16,), jnp.int32),
            pltpu.VMEM((_B, _HSL, _W), jnp.float32),
            pltpu.SemaphoreType.DMA,
        ],
    )(table, idx)


def _tc_loss_body(logits_ref, targets_ref, w_ref, out_ref):
    b = pl.program_id(0)
    r = pl.program_id(1)
    l = logits_ref[0]
    t = targets_ref[0]
    w = w_ref[0]
    m = jnp.max(l, axis=0)
    e = jnp.exp(l - m[None])
    s = jnp.sum(e, axis=0)
    cls = lax.broadcasted_iota(jnp.int32, l.shape, 0)
    lt = jnp.sum(jnp.where(cls == t[None], l, 0.0), axis=0)
    log_yg = (lt - m) - jnp.log(s)
    pow_q = jnp.exp(_Q * log_yg)
    term = (1.0 - pow_q) * (1.0 / _Q) - _C
    partial = jnp.sum(term * w) * (1.0 / _N)

    @pl.when((b == 0) & (r == 0))
    def _init():
        out_ref[0, 0] = 0.0

    out_ref[0, 0] += partial


@functools.partial(jax.jit, static_argnames=("block_r",))
def _tc_loss(logits, targets, w16, block_r=256):
    nr = _H // block_r
    return pl.pallas_call(
        _tc_loss_body,
        grid=(_B, nr),
        in_specs=[
            pl.BlockSpec((1, _NCLS, block_r, _W), lambda b, r: (b, 0, r, 0)),
            pl.BlockSpec((1, block_r, _W), lambda b, r: (b, r, 0)),
            pl.BlockSpec((1, block_r, _W), lambda b, r: (b, r, 0)),
        ],
        out_specs=pl.BlockSpec((1, 1), lambda b, r: (0, 0),
                               memory_space=pltpu.SMEM),
        out_shape=jax.ShapeDtypeStruct((1, 1), jnp.float32),
    )(logits, targets, w16)


def kernel(logits, weight, targets, indexes):
    w16 = _sc_gather(weight.reshape(_ROWS, _H, _W), indexes)
    out = _tc_loss(logits, targets.reshape(_B, _H, _W), w16)
    return out[0, 0]


# speedup vs baseline: 1.1364x; 1.0281x over previous
import functools

import jax
import jax.numpy as jnp
from jax import lax
from jax.experimental import pallas as pl
from jax.experimental.pallas import tpu as pltpu
from jax.experimental.pallas import tpu_sc as plsc

_Q = 0.7
_K = 0.8
_C = (1.0 - _K**_Q) / _Q

_B = 16
_NCLS = 21
_H = 256
_W = 256
_ROWS = 2048
_N = _B * _H * _W

_NC = 2
_NS = 16
_NW = _NC * _NS
_CHUNK = 2048
_CPR = (_H * _W) // _CHUNK


_HSL = _H // _NW


def _sc_gather_body(table_ref, idx_ref, out_ref, idx_v, buf_v, sem):
    wid = lax.axis_index("s") * _NC + lax.axis_index("c")
    pltpu.sync_copy(idx_ref, idx_v)
    h0 = wid * _HSL
    pltpu.async_copy(table_ref.at[idx_v, pl.ds(h0, _HSL)], buf_v, sem).wait()
    pltpu.sync_copy(buf_v, out_ref.at[:, pl.ds(h0, _HSL)])


@jax.jit
def _sc_gather(table, idx):
    mesh = plsc.VectorSubcoreMesh(
        core_axis_name="c", subcore_axis_name="s",
        num_cores=_NC, num_subcores=_NS)
    return pl.kernel(
        _sc_gather_body,
        out_type=jax.ShapeDtypeStruct((_B, _H, _W), jnp.float32),
        mesh=mesh,
        scratch_types=[
            pltpu.VMEM((16,), jnp.int32),
            pltpu.VMEM((_B, _HSL, _W), jnp.float32),
            pltpu.SemaphoreType.DMA,
        ],
    )(table, idx)


def _tc_loss_body(logits_ref, targets_ref, w_ref, out_ref):
    step = pl.program_id(0)
    l = logits_ref[...]
    t = targets_ref[...]
    w = w_ref[...]
    m = jnp.max(l, axis=1)
    e = jnp.exp(l - m[:, None])
    s = jnp.sum(e, axis=1)
    cls = lax.broadcasted_iota(jnp.int32, l.shape, 1)
    lt = jnp.sum(jnp.where(cls == t[:, None], l, 0.0), axis=1)
    log_yg = (lt - m) - jnp.log(s)
    pow_q = jnp.exp(_Q * log_yg)
    term = (1.0 - pow_q) * (1.0 / _Q) - _C
    partial = jnp.sum(term * w) * (1.0 / _N)

    @pl.when(step == 0)
    def _init():
        out_ref[0, 0] = 0.0

    out_ref[0, 0] += partial


@functools.partial(jax.jit, static_argnames=("block_b",))
def _tc_loss(logits, targets, w16, block_b=2):
    nsteps = _B // block_b
    return pl.pallas_call(
        _tc_loss_body,
        grid=(nsteps,),
        in_specs=[
            pl.BlockSpec((block_b, _NCLS, _H, _W), lambda b: (b, 0, 0, 0)),
            pl.BlockSpec((block_b, _H, _W), lambda b: (b, 0, 0)),
            pl.BlockSpec((block_b, _H, _W), lambda b: (b, 0, 0)),
        ],
        out_specs=pl.BlockSpec((1, 1), lambda b: (0, 0),
                               memory_space=pltpu.SMEM),
        out_shape=jax.ShapeDtypeStruct((1, 1), jnp.float32),
    )(logits, targets, w16)


def kernel(logits, weight, targets, indexes):
    w16 = _sc_gather(weight.reshape(_ROWS, _H, _W), indexes)
    out = _tc_loss(logits, targets.reshape(_B, _H, _W), w16)
    return out[0, 0]


# speedup vs baseline: 1.1647x; 1.0249x over previous
import functools

import jax
import jax.numpy as jnp
from jax import lax
from jax.experimental import pallas as pl
from jax.experimental.pallas import tpu as pltpu
from jax.experimental.pallas import tpu_sc as plsc

_Q = 0.7
_K = 0.8
_C = (1.0 - _K**_Q) / _Q

_B = 16
_NCLS = 21
_H = 256
_W = 256
_ROWS = 2048
_N = _B * _H * _W

_NC = 2
_NS = 16
_NW = _NC * _NS
_CHUNK = 2048
_CPR = (_H * _W) // _CHUNK


_HSL = _H // _NW


def _sc_gather_body(table_ref, idx_ref, out_ref, idx_v, buf_v, sem_a, sem_b):
    wid = lax.axis_index("s") * _NC + lax.axis_index("c")
    pltpu.sync_copy(idx_ref, idx_v)
    h0 = wid * _HSL
    ga = pltpu.async_copy(
        table_ref.at[idx_v.at[pl.ds(0, _B // 2)], pl.ds(h0, _HSL)],
        buf_v.at[pl.ds(0, _B // 2)], sem_a)
    gb = pltpu.async_copy(
        table_ref.at[idx_v.at[pl.ds(_B // 2, _B // 2)], pl.ds(h0, _HSL)],
        buf_v.at[pl.ds(_B // 2, _B // 2)], sem_b)
    ga.wait()
    wa = pltpu.async_copy(buf_v.at[pl.ds(0, _B // 2)],
                          out_ref.at[pl.ds(0, _B // 2), pl.ds(h0, _HSL)], sem_a)
    gb.wait()
    wb = pltpu.async_copy(buf_v.at[pl.ds(_B // 2, _B // 2)],
                          out_ref.at[pl.ds(_B // 2, _B // 2), pl.ds(h0, _HSL)],
                          sem_b)
    wa.wait()
    wb.wait()


@jax.jit
def _sc_gather(table, idx):
    mesh = plsc.VectorSubcoreMesh(
        core_axis_name="c", subcore_axis_name="s",
        num_cores=_NC, num_subcores=_NS)
    return pl.kernel(
        _sc_gather_body,
        out_type=jax.ShapeDtypeStruct((_B, _H, _W), jnp.float32),
        mesh=mesh,
        scratch_types=[
            pltpu.VMEM((16,), jnp.int32),
            pltpu.VMEM((_B, _HSL, _W), jnp.float32),
            pltpu.SemaphoreType.DMA,
            pltpu.SemaphoreType.DMA,
        ],
    )(table, idx)


def _tc_loss_body(logits_ref, targets_ref, w_ref, out_ref):
    step = pl.program_id(0)
    l = logits_ref[...]
    t = targets_ref[...]
    w = w_ref[...]
    m = jnp.max(l, axis=1)
    e = jnp.exp(l - m[:, None])
    s = jnp.sum(e, axis=1)
    cls = lax.broadcasted_iota(jnp.int32, l.shape, 1)
    lt = jnp.sum(jnp.where(cls == t[:, None], l, 0.0), axis=1)
    log_yg = (lt - m) - jnp.log(s)
    pow_q = jnp.exp(_Q * log_yg)
    term = (1.0 - pow_q) * (1.0 / _Q) - _C
    partial = jnp.sum(term * w) * (1.0 / _N)

    @pl.when(step == 0)
    def _init():
        out_ref[0, 0] = 0.0

    out_ref[0, 0] += partial


@functools.partial(jax.jit, static_argnames=("block_b",))
def _tc_loss(logits, targets, w16, block_b=2):
    nsteps = _B // block_b
    return pl.pallas_call(
        _tc_loss_body,
        grid=(nsteps,),
        in_specs=[
            pl.BlockSpec((block_b, _NCLS, _H, _W), lambda b: (b, 0, 0, 0)),
            pl.BlockSpec((block_b, _H, _W), lambda b: (b, 0, 0)),
            pl.BlockSpec((block_b, _H, _W), lambda b: (b, 0, 0)),
        ],
        out_specs=pl.BlockSpec((1, 1), lambda b: (0, 0),
                               memory_space=pltpu.SMEM),
        out_shape=jax.ShapeDtypeStruct((1, 1), jnp.float32),
    )(logits, targets, w16)


def kernel(logits, weight, targets, indexes):
    w16 = _sc_gather(weight.reshape(_ROWS, _H, _W), indexes)
    out = _tc_loss(logits, targets.reshape(_B, _H, _W), w16)
    return out[0, 0]
